# Initial kernel scaffold; baseline (speedup 1.0000x reference)
import jax
import jax.numpy as jnp
from jax.experimental import pallas as pl


def kernel(x, queries, memories, conv1_w, conv1_b, conv2_w, conv2_b, fc1_w, fc1_b, sq_w, sq_b, sk_w, sk_b, to_q_w, to_kv_w, to_out_w, to_out_b, fc2_w, fc2_b, mask):
    raise NotImplementedError("write your pallas kernel here")


# speedup vs baseline: 1.7197x; 1.7197x over previous
import numpy as np
import jax
import jax.numpy as jnp
from jax.experimental import pallas as pl
from jax.experimental.pallas import tpu as pltpu

HIDDEN = 512
HEADS = 8
DHEAD = 64
TOPK = 8
CHUNK = 32
NQ = 128
NCHUNK = 625
G = 4
STEPS = NQ // G
ROWS = G * HEADS
SEL = G * TOPK
TOK = SEL * CHUNK
NEG = -3.4028235e38
HIGHEST = jax.lax.Precision.HIGHEST
HIGH = jax.lax.Precision.HIGH


def _bfdot(a, b):
    return jax.lax.dot_general(
        a.astype(jnp.bfloat16), b.astype(jnp.bfloat16),
        (((1,), (0,)), ((), ())), preferred_element_type=jnp.float32)


def _bfdot_t(a, b):
    return jax.lax.dot_general(
        a.astype(jnp.bfloat16), b.astype(jnp.bfloat16),
        (((1,), (1,)), ((), ())), preferred_element_type=jnp.float32)


def _score_kernel(mem3, q2, sqw, sqb, skw, skb, tqw, wkt, pos,
                  qk_out, qkpos_out, idx_out, w_out):
    acc = mem3[:, 0, :]
    for j in range(1, CHUNK):
        acc = acc + mem3[:, j, :]
    summar = acc / (np.float32(CHUNK) + np.float32(1e-5))

    q2v = q2[...]
    sq = _bfdot(q2v, sqw[...]) + sqb[...]
    sk = _bfdot(summar, skw[...]) + skb[...]
    sim = _bfdot_t(sq, sk) * np.float32(HIDDEN ** -0.5)

    iota = jax.lax.broadcasted_iota(jnp.int32, (NQ, NCHUNK), 1)
    s = sim
    logits = []
    idxs = []
    for _ in range(TOPK):
        m = jnp.max(s, axis=1, keepdims=True)
        cand = jnp.where(s >= m, iota, jnp.int32(2 ** 30))
        ix = jnp.min(cand, axis=1, keepdims=True)
        s = jnp.where(iota == ix, jnp.float32(NEG), s)
        logits.append(m)
        idxs.append(ix)
    lg = jnp.concatenate(logits, axis=1)
    e = jnp.exp(lg - lg[:, 0:1])
    w_out[...] = e / jnp.sum(e, axis=1, keepdims=True)
    idx_out[...] = jnp.concatenate(idxs, axis=1)

    qall = _bfdot(q2v, tqw[...]) * np.float32(DHEAD ** -0.5)
    wktv = wkt[...]
    posv = pos[...]
    for h in range(HEADS):
        qk_h = _bfdot(qall[:, h * DHEAD:(h + 1) * DHEAD],
                      wktv[h * DHEAD:(h + 1) * DHEAD, :])
        qk_out[:, h, :] = qk_h
        qkpos_out[:, h, :] = _bfdot_t(qk_h, posv)


def _attn_kernel(idx_ref, memhbm, qk_ref, qkpos_ref, w_ref, pos_ref,
                 seg_ref, segt_ref, tc_ref, tct_ref,
                 wv_ref, wo_ref, wob_ref, f2w_ref, f2b_ref,
                 out_ref, memv, stk, o1acc, sem, gsem):
    s = pl.program_id(0)

    def issue(step, slot):
        for t in range(SEL):
            c = idx_ref[step * SEL + t]
            pltpu.make_async_copy(
                memv.at[pl.ds(c * CHUNK, CHUNK), :],
                stk.at[slot, pl.ds(t * CHUNK, CHUNK), :],
                gsem.at[slot]).start()

    @pl.when(s == 0)
    def _load():
        dma = pltpu.make_async_copy(memhbm, memv, sem)
        dma.start()
        dma.wait()
        issue(0, 0)

    @pl.when(s < STEPS - 1)
    def _prefetch():
        issue(s + 1, (s + 1) % 2)

    slot = s % 2
    for t in range(SEL):
        pltpu.make_async_copy(
            memv.at[pl.ds(0, CHUNK), :],
            stk.at[0, pl.ds(t * CHUNK, CHUNK), :],
            gsem.at[slot]).wait()

    st = stk[slot]
    qk = qk_ref[...]
    segv = seg_ref[...]
    tcv = tc_ref[...]
    qkpos_t = _bfdot(qkpos_ref[...], tct_ref[...])
    scores = jax.lax.dot_general(qk, st, (((1,), (1,)), ((), ())),
                                 precision=HIGHEST) + qkpos_t
    rowmax = jnp.max(scores, axis=1, keepdims=True)
    ex = jnp.exp(scores - rowmax)
    denom = _bfdot(ex, segv)

    ri = jax.lax.broadcasted_iota(jnp.int32, (ROWS, SEL), 0)
    ci = jax.lax.broadcasted_iota(jnp.int32, (ROWS, SEL), 1)
    same_g = (ri // HEADS) == (ci // TOPK)
    wrow = jnp.broadcast_to(w_ref[...].reshape(1, SEL), (ROWS, SEL))
    fac = jnp.where(same_g, wrow / jnp.maximum(denom, 1e-20), 0.0)
    spread = _bfdot(fac, segt_ref[...])
    attnw = ex * spread
    colsum = _bfdot(attnw, tcv)
    pool = (jnp.dot(attnw, st, precision=HIGHEST)
            + _bfdot(colsum, pos_ref[...]))

    z = _bfdot(pool, wv_ref[...])
    r2 = jax.lax.broadcasted_iota(jnp.int32, (ROWS, HIDDEN), 0)
    c2 = jax.lax.broadcasted_iota(jnp.int32, (ROWS, HIDDEN), 1)
    keep = (r2 % HEADS) == (c2 // DHEAD)
    o1 = jnp.where(keep, z, 0.0).reshape(G, HEADS, HIDDEN).sum(axis=1)
    o1acc[s, pl.ds(0, G), :] = o1

    @pl.when(s == STEPS - 1)
    def _finish():
        o1full = o1acc[...][:, 0:G, :].reshape(NQ, HIDDEN)
        htm = _bfdot(o1full, wo_ref[...]) + wob_ref[...]
        out_ref[...] = _bfdot(htm, f2w_ref[...]) + f2b_ref[...]


def _pos_emb():
    freqs = np.arange(0, HIDDEN, 2.0)
    inv = 10000.0 ** (-freqs / HIDDEN)
    seq = np.arange(CHUNK - 1, -1, -1.0)
    si = seq[:, None] * inv[None, :]
    return np.concatenate([np.sin(si), np.cos(si)], axis=-1).astype(np.float32)


@jax.jit
def _run(queries, memories, sq_w, sq_b, sk_w, sk_b, to_q_w, to_kv_w,
         to_out_w, to_out_b, fc2_w, fc2_b):
    q2 = queries[0]
    mem3 = memories[0].reshape(NCHUNK, CHUNK, HIDDEN)
    mem2 = memories[0]
    wkt = to_kv_w[:, :HIDDEN].T
    wv = to_kv_w[:, HIDDEN:]
    pos = jnp.asarray(_pos_emb())

    qk, qkpos, idx8, w8 = pl.pallas_call(
        _score_kernel,
        out_shape=[
            jax.ShapeDtypeStruct((NQ, HEADS, HIDDEN), jnp.float32),
            jax.ShapeDtypeStruct((NQ, HEADS, CHUNK), jnp.float32),
            jax.ShapeDtypeStruct((NQ, TOPK), jnp.int32),
            jax.ShapeDtypeStruct((NQ, TOPK), jnp.float32),
        ],
    )(mem3, q2, sq_w, sq_b.reshape(1, HIDDEN), sk_w, sk_b.reshape(1, HIDDEN),
      to_q_w, wkt, pos)

    qk2 = qk.reshape(NQ * HEADS, HIDDEN)
    qkpos2 = qkpos.reshape(NQ * HEADS, CHUNK)
    idx_flat = idx8.reshape(NQ * TOPK)
    w3 = w8.reshape(STEPS, 1, SEL)

    seg = jnp.asarray(
        (np.arange(TOK)[:, None] // CHUNK == np.arange(SEL)[None, :]
         ).astype(np.float32))
    tc = jnp.asarray(
        (np.arange(TOK)[:, None] % CHUNK == np.arange(CHUNK)[None, :]
         ).astype(np.float32))
    segt = seg.T
    tct = tc.T

    grid_spec = pltpu.PrefetchScalarGridSpec(
        num_scalar_prefetch=1,
        grid=(STEPS,),
        in_specs=[
            pl.BlockSpec(memory_space=pl.ANY),
            pl.BlockSpec((ROWS, HIDDEN), lambda s, n: (s, 0)),
            pl.BlockSpec((ROWS, CHUNK), lambda s, n: (s, 0)),
            pl.BlockSpec((1, 1, SEL), lambda s, n: (s, 0, 0)),
            pl.BlockSpec((CHUNK, HIDDEN), lambda s, n: (0, 0)),
            pl.BlockSpec((TOK, SEL), lambda s, n: (0, 0)),
            pl.BlockSpec((SEL, TOK), lambda s, n: (0, 0)),
            pl.BlockSpec((TOK, CHUNK), lambda s, n: (0, 0)),
            pl.BlockSpec((CHUNK, TOK), lambda s, n: (0, 0)),
            pl.BlockSpec((HIDDEN, HIDDEN), lambda s, n: (0, 0)),
            pl.BlockSpec((HIDDEN, HIDDEN), lambda s, n: (0, 0)),
            pl.BlockSpec((1, HIDDEN), lambda s, n: (0, 0)),
            pl.BlockSpec((HIDDEN, 5), lambda s, n: (0, 0)),
            pl.BlockSpec((1, 5), lambda s, n: (0, 0)),
        ],
        out_specs=pl.BlockSpec((NQ, 5), lambda s, n: (0, 0)),
        scratch_shapes=[
            pltpu.VMEM((NCHUNK * CHUNK, HIDDEN), jnp.float32),
            pltpu.VMEM((2, TOK, HIDDEN), jnp.float32),
            pltpu.VMEM((STEPS, 8, HIDDEN), jnp.float32),
            pltpu.SemaphoreType.DMA,
            pltpu.SemaphoreType.DMA((2,)),
        ],
    )

    out = pl.pallas_call(
        _attn_kernel,
        grid_spec=grid_spec,
        out_shape=jax.ShapeDtypeStruct((NQ, 5), jnp.float32),
    )(idx_flat, mem2, qk2, qkpos2, w3, pos, seg, segt, tc, tct, wv, to_out_w,
      to_out_b.reshape(1, HIDDEN), fc2_w, fc2_b.reshape(1, 5))

    return out.reshape(1, NQ, 5)


def kernel(x, queries, memories, conv1_w, conv1_b, conv2_w, conv2_b,
           fc1_w, fc1_b, sq_w, sq_b, sk_w, sk_b, to_q_w, to_kv_w,
           to_out_w, to_out_b, fc2_w, fc2_b, mask):
    return _run(queries, memories, sq_w, sq_b, sk_w, sk_b, to_q_w, to_kv_w,
                to_out_w, to_out_b, fc2_w, fc2_b)


# speedup vs baseline: 5.0244x; 2.9216x over previous
import numpy as np
import jax
import jax.numpy as jnp
from jax.experimental import pallas as pl
from jax.experimental.pallas import tpu as pltpu

HIDDEN = 512
HEADS = 8
DHEAD = 64
TOPK = 8
CHUNK = 32
NQ = 128
NCHUNK = 625
G = 4
STEPS = NQ // G
ROWS = G * HEADS
SEL = G * TOPK
TOK = SEL * CHUNK
NEG = -3.4028235e38
HIGHEST = jax.lax.Precision.HIGHEST
HIGH = jax.lax.Precision.HIGH


def _bfdot(a, b):
    return jax.lax.dot_general(
        a.astype(jnp.bfloat16), b.astype(jnp.bfloat16),
        (((1,), (0,)), ((), ())), preferred_element_type=jnp.float32)


def _bfdot_t(a, b):
    return jax.lax.dot_general(
        a.astype(jnp.bfloat16), b.astype(jnp.bfloat16),
        (((1,), (1,)), ((), ())), preferred_element_type=jnp.float32)


def _score_kernel(mem3, q2, sqw, sqb, skw, skb, tqw, wkt, pos,
                  qk_out, qkpos_out, idx_out, w_out):
    acc0 = mem3[:, 0, :]
    acc1 = mem3[:, 1, :]
    for j in range(2, CHUNK, 2):
        acc0 = acc0 + mem3[:, j, :]
        acc1 = acc1 + mem3[:, j + 1, :]
    summar = (acc0 + acc1) / (np.float32(CHUNK) + np.float32(1e-5))

    q2v = q2[...]
    sq = _bfdot(q2v, sqw[...]) + sqb[...]
    sk = _bfdot(summar, skw[...]) + skb[...]
    sim = _bfdot_t(sq, sk) * np.float32(HIDDEN ** -0.5)

    iota = jax.lax.broadcasted_iota(jnp.int32, (NQ, NCHUNK), 1)
    s = sim
    logits = []
    idxs = []
    for _ in range(TOPK):
        m = jnp.max(s, axis=1, keepdims=True)
        cand = jnp.where(s >= m, iota, jnp.int32(2 ** 30))
        ix = jnp.min(cand, axis=1, keepdims=True)
        s = jnp.where(iota == ix, jnp.float32(NEG), s)
        logits.append(m)
        idxs.append(ix)
    lg = jnp.concatenate(logits, axis=1)
    e = jnp.exp(lg - lg[:, 0:1])
    w_out[...] = e / jnp.sum(e, axis=1, keepdims=True)
    idx_out[...] = jnp.concatenate(idxs, axis=1)

    qall = _bfdot(q2v, tqw[...]) * np.float32(DHEAD ** -0.5)
    wktv = wkt[...]
    posv = pos[...]
    for h in range(HEADS):
        qk_h = _bfdot(qall[:, h * DHEAD:(h + 1) * DHEAD],
                      wktv[h * DHEAD:(h + 1) * DHEAD, :])
        qk_out[:, h, :] = qk_h
        qkpos_out[:, h, :] = _bfdot_t(qk_h, posv)


def _attn_kernel(idx_ref, memhbm, qk_ref, qkpos_ref, w_ref, pos_ref,
                 seg_ref, segt_ref, tc_ref, tct_ref,
                 wv_ref, wo_ref, wob_ref, f2w_ref, f2b_ref,
                 out_ref, stk, o1acc, gsem):
    s = pl.program_id(0)

    def issue(step, slot):
        for t in range(SEL):
            c = idx_ref[step * SEL + t]
            pltpu.make_async_copy(
                memhbm.at[pl.ds(c * CHUNK, CHUNK), :],
                stk.at[slot, pl.ds(t * CHUNK, CHUNK), :],
                gsem.at[slot]).start()

    @pl.when(s == 0)
    def _first():
        issue(0, 0)

    @pl.when(s < STEPS - 1)
    def _prefetch():
        issue(s + 1, (s + 1) % 2)

    slot = s % 2
    for t in range(SEL):
        pltpu.make_async_copy(
            memhbm.at[pl.ds(0, CHUNK), :],
            stk.at[0, pl.ds(t * CHUNK, CHUNK), :],
            gsem.at[slot]).wait()

    st = stk[slot]
    qk = qk_ref[...]
    segv = seg_ref[...]
    tcv = tc_ref[...]
    qkpos_t = _bfdot(qkpos_ref[...], tct_ref[...])
    scores = _bfdot_t(qk, st) + qkpos_t
    rowmax = jnp.max(scores, axis=1, keepdims=True)
    ex = jnp.exp(scores - rowmax)
    denom = _bfdot(ex, segv)

    ri = jax.lax.broadcasted_iota(jnp.int32, (ROWS, SEL), 0)
    ci = jax.lax.broadcasted_iota(jnp.int32, (ROWS, SEL), 1)
    same_g = (ri // HEADS) == (ci // TOPK)
    wrow = jnp.broadcast_to(w_ref[...].reshape(1, SEL), (ROWS, SEL))
    fac = jnp.where(same_g, wrow / jnp.maximum(denom, 1e-20), 0.0)
    spread = _bfdot(fac, segt_ref[...])
    attnw = ex * spread
    colsum = _bfdot(attnw, tcv)
    pool = (_bfdot(attnw, st)
            + _bfdot(colsum, pos_ref[...]))

    z = _bfdot(pool, wv_ref[...])
    r2 = jax.lax.broadcasted_iota(jnp.int32, (ROWS, HIDDEN), 0)
    c2 = jax.lax.broadcasted_iota(jnp.int32, (ROWS, HIDDEN), 1)
    keep = (r2 % HEADS) == (c2 // DHEAD)
    o1 = jnp.where(keep, z, 0.0).reshape(G, HEADS, HIDDEN).sum(axis=1)
    o1acc[s, pl.ds(0, G), :] = o1

    @pl.when(s == STEPS - 1)
    def _finish():
        o1full = o1acc[...][:, 0:G, :].reshape(NQ, HIDDEN)
        htm = _bfdot(o1full, wo_ref[...]) + wob_ref[...]
        out_ref[...] = _bfdot(htm, f2w_ref[...]) + f2b_ref[...]


def _pos_emb():
    freqs = np.arange(0, HIDDEN, 2.0)
    inv = 10000.0 ** (-freqs / HIDDEN)
    seq = np.arange(CHUNK - 1, -1, -1.0)
    si = seq[:, None] * inv[None, :]
    return np.concatenate([np.sin(si), np.cos(si)], axis=-1).astype(np.float32)


@jax.jit
def _run(queries, memories, sq_w, sq_b, sk_w, sk_b, to_q_w, to_kv_w,
         to_out_w, to_out_b, fc2_w, fc2_b):
    q2 = queries[0]
    mem3 = memories[0].reshape(NCHUNK, CHUNK, HIDDEN)
    mem2 = memories[0]
    wkt = to_kv_w[:, :HIDDEN].T
    wv = to_kv_w[:, HIDDEN:]
    pos = jnp.asarray(_pos_emb())

    qk, qkpos, idx8, w8 = pl.pallas_call(
        _score_kernel,
        out_shape=[
            jax.ShapeDtypeStruct((NQ, HEADS, HIDDEN), jnp.float32),
            jax.ShapeDtypeStruct((NQ, HEADS, CHUNK), jnp.float32),
            jax.ShapeDtypeStruct((NQ, TOPK), jnp.int32),
            jax.ShapeDtypeStruct((NQ, TOPK), jnp.float32),
        ],
    )(mem3, q2, sq_w, sq_b.reshape(1, HIDDEN), sk_w, sk_b.reshape(1, HIDDEN),
      to_q_w, wkt, pos)

    qk2 = qk.reshape(NQ * HEADS, HIDDEN)
    qkpos2 = qkpos.reshape(NQ * HEADS, CHUNK)
    idx_flat = idx8.reshape(NQ * TOPK)
    w3 = w8.reshape(STEPS, 1, SEL)

    seg = jnp.asarray(
        (np.arange(TOK)[:, None] // CHUNK == np.arange(SEL)[None, :]
         ).astype(np.float32))
    tc = jnp.asarray(
        (np.arange(TOK)[:, None] % CHUNK == np.arange(CHUNK)[None, :]
         ).astype(np.float32))
    segt = seg.T
    tct = tc.T

    grid_spec = pltpu.PrefetchScalarGridSpec(
        num_scalar_prefetch=1,
        grid=(STEPS,),
        in_specs=[
            pl.BlockSpec(memory_space=pl.ANY),
            pl.BlockSpec((ROWS, HIDDEN), lambda s, n: (s, 0)),
            pl.BlockSpec((ROWS, CHUNK), lambda s, n: (s, 0)),
            pl.BlockSpec((1, 1, SEL), lambda s, n: (s, 0, 0)),
            pl.BlockSpec((CHUNK, HIDDEN), lambda s, n: (0, 0)),
            pl.BlockSpec((TOK, SEL), lambda s, n: (0, 0)),
            pl.BlockSpec((SEL, TOK), lambda s, n: (0, 0)),
            pl.BlockSpec((TOK, CHUNK), lambda s, n: (0, 0)),
            pl.BlockSpec((CHUNK, TOK), lambda s, n: (0, 0)),
            pl.BlockSpec((HIDDEN, HIDDEN), lambda s, n: (0, 0)),
            pl.BlockSpec((HIDDEN, HIDDEN), lambda s, n: (0, 0)),
            pl.BlockSpec((1, HIDDEN), lambda s, n: (0, 0)),
            pl.BlockSpec((HIDDEN, 5), lambda s, n: (0, 0)),
            pl.BlockSpec((1, 5), lambda s, n: (0, 0)),
        ],
        out_specs=pl.BlockSpec((NQ, 5), lambda s, n: (0, 0)),
        scratch_shapes=[
            pltpu.VMEM((2, TOK, HIDDEN), jnp.float32),
            pltpu.VMEM((STEPS, 8, HIDDEN), jnp.float32),
            pltpu.SemaphoreType.DMA((2,)),
        ],
    )

    out = pl.pallas_call(
        _attn_kernel,
        grid_spec=grid_spec,
        out_shape=jax.ShapeDtypeStruct((NQ, 5), jnp.float32),
    )(idx_flat, mem2, qk2, qkpos2, w3, pos, seg, segt, tc, tct, wv, to_out_w,
      to_out_b.reshape(1, HIDDEN), fc2_w, fc2_b.reshape(1, 5))

    return out.reshape(1, NQ, 5)


def kernel(x, queries, memories, conv1_w, conv1_b, conv2_w, conv2_b,
           fc1_w, fc1_b, sq_w, sq_b, sk_w, sk_b, to_q_w, to_kv_w,
           to_out_w, to_out_b, fc2_w, fc2_b, mask):
    return _run(queries, memories, sq_w, sq_b, sk_w, sk_b, to_q_w, to_kv_w,
                to_out_w, to_out_b, fc2_w, fc2_b)
